# baseline (speedup 1.0000x reference)
import functools

import jax
import jax.numpy as jnp
from jax import lax
from jax.experimental import pallas as pl
from jax.experimental.pallas import tpu as pltpu
from jax.experimental.pallas import tpu_sc as plsc

N_TOTAL = 100000
L = 16
NS = 16

BLK = 4096
N_TC = 49152
N_SC = N_TOTAL - N_TC
CH = 3200

_mesh = plsc.VectorSubcoreMesh(core_axis_name="c", subcore_axis_name="s",
                               num_cores=1)


@functools.partial(
    pl.kernel,
    out_type=jax.ShapeDtypeStruct((N_SC,), jnp.float32),
    mesh=_mesh,
    scratch_types=[
        pltpu.VMEM((CH,), jnp.float32),
        pltpu.VMEM((CH,), jnp.int32),
        pltpu.VMEM((CH,), jnp.float32),
        pltpu.VMEM((L,), jnp.float32),
        pltpu.VMEM((L,), jnp.float32),
        pltpu.SemaphoreType.DMA,
        pltpu.SemaphoreType.DMA,
        pltpu.SemaphoreType.DMA,
    ],
)
def _rescale_sc(x_hbm, t_hbm, shift_hbm, scale_hbm, out_hbm,
                x_v, t_v, o_v, sc_v, sh_v, sem_tab, sem_x, sem_t):
    wid = lax.axis_index("s")
    base = jnp.minimum(wid * CH, N_SC - CH)

    c_sc = pltpu.async_copy(scale_hbm, sc_v, sem_tab)
    c_sh = pltpu.async_copy(shift_hbm, sh_v, sem_tab)
    c_x = pltpu.async_copy(x_hbm.at[pl.ds(N_TC + base, CH)], x_v, sem_x)
    c_t = pltpu.async_copy(t_hbm.at[pl.ds(N_TC + base, CH)], t_v, sem_t)
    c_sc.wait()
    c_sh.wait()
    scale_reg = sc_v[...]
    shift_reg = sh_v[...]
    c_x.wait()
    c_t.wait()

    @plsc.parallel_loop(0, CH, step=L, unroll=8)
    def _(off):
        t = t_v[pl.ds(off, L)]
        xv = x_v[pl.ds(off, L)]
        s = scale_reg.at[t].get(mode="promise_in_bounds")
        b = shift_reg.at[t].get(mode="promise_in_bounds")
        o_v[pl.ds(off, L)] = xv * s + b

    pltpu.sync_copy(o_v, out_hbm.at[pl.ds(base, CH)])


def _rescale_tc_body(x_ref, t_ref, shift_ref, scale_ref, o_ref):
    t = t_ref[...]
    xv = x_ref[...]
    s = jnp.full_like(xv, scale_ref[0])
    b = jnp.full_like(xv, shift_ref[0])
    for k in range(1, L):
        m = t == k
        s = jnp.where(m, scale_ref[k], s)
        b = jnp.where(m, shift_ref[k], b)
    o_ref[...] = xv * s + b


_rescale_tc = pl.pallas_call(
    _rescale_tc_body,
    grid=(N_TC // BLK,),
    in_specs=[
        pl.BlockSpec((BLK,), lambda i: (i,)),
        pl.BlockSpec((BLK,), lambda i: (i,)),
        pl.BlockSpec(memory_space=pltpu.SMEM),
        pl.BlockSpec(memory_space=pltpu.SMEM),
    ],
    out_specs=pl.BlockSpec((BLK,), lambda i: (i,)),
    out_shape=jax.ShapeDtypeStruct((N_TC,), jnp.float32),
)


def kernel(x, atom_type, shift, scale):
    xf = x.reshape(-1)
    t = atom_type.astype(jnp.int32)
    sc_out = _rescale_sc(xf, t, shift, scale)
    tc_out = _rescale_tc(xf[:N_TC], t[:N_TC], shift, scale)
    out = jnp.concatenate([tc_out, sc_out])
    return out.reshape(N_TOTAL, 1)


# speedup vs baseline: 1.3222x; 1.3222x over previous
import functools

import jax
import jax.numpy as jnp
from jax import lax
from jax.experimental import pallas as pl
from jax.experimental.pallas import tpu as pltpu
from jax.experimental.pallas import tpu_sc as plsc

N_TOTAL = 100000
L = 16
NS = 16

BLK = 49152
N_TC = 49152
N_SC = N_TOTAL - N_TC
CH = 3200

_mesh = plsc.VectorSubcoreMesh(core_axis_name="c", subcore_axis_name="s",
                               num_cores=1)


@functools.partial(
    pl.kernel,
    out_type=jax.ShapeDtypeStruct((N_SC,), jnp.float32),
    mesh=_mesh,
    scratch_types=[
        pltpu.VMEM((CH,), jnp.float32),
        pltpu.VMEM((CH,), jnp.int32),
        pltpu.VMEM((CH,), jnp.float32),
        pltpu.VMEM((L,), jnp.float32),
        pltpu.VMEM((L,), jnp.float32),
        pltpu.SemaphoreType.DMA,
        pltpu.SemaphoreType.DMA,
        pltpu.SemaphoreType.DMA,
    ],
)
def _rescale_sc(x_hbm, t_hbm, shift_hbm, scale_hbm, out_hbm,
                x_v, t_v, o_v, sc_v, sh_v, sem_tab, sem_x, sem_t):
    wid = lax.axis_index("s")
    base = jnp.minimum(wid * CH, N_SC - CH)

    c_sc = pltpu.async_copy(scale_hbm, sc_v, sem_tab)
    c_sh = pltpu.async_copy(shift_hbm, sh_v, sem_tab)
    c_x = pltpu.async_copy(x_hbm.at[pl.ds(N_TC + base, CH)], x_v, sem_x)
    c_t = pltpu.async_copy(t_hbm.at[pl.ds(N_TC + base, CH)], t_v, sem_t)
    c_sc.wait()
    c_sh.wait()
    scale_reg = sc_v[...]
    shift_reg = sh_v[...]
    c_x.wait()
    c_t.wait()

    @plsc.parallel_loop(0, CH, step=L, unroll=8)
    def _(off):
        t = t_v[pl.ds(off, L)]
        xv = x_v[pl.ds(off, L)]
        s = scale_reg.at[t].get(mode="promise_in_bounds")
        b = shift_reg.at[t].get(mode="promise_in_bounds")
        o_v[pl.ds(off, L)] = xv * s + b

    pltpu.sync_copy(o_v, out_hbm.at[pl.ds(base, CH)])


def _rescale_tc_body(x_ref, t_ref, shift_ref, scale_ref, o_ref):
    t = t_ref[...]
    xv = x_ref[...]
    s = jnp.full_like(xv, scale_ref[0])
    b = jnp.full_like(xv, shift_ref[0])
    for k in range(1, L):
        m = t == k
        s = jnp.where(m, scale_ref[k], s)
        b = jnp.where(m, shift_ref[k], b)
    o_ref[...] = xv * s + b


_rescale_tc = pl.pallas_call(
    _rescale_tc_body,
    grid=(N_TC // BLK,),
    in_specs=[
        pl.BlockSpec((BLK,), lambda i: (i,)),
        pl.BlockSpec((BLK,), lambda i: (i,)),
        pl.BlockSpec(memory_space=pltpu.SMEM),
        pl.BlockSpec(memory_space=pltpu.SMEM),
    ],
    out_specs=pl.BlockSpec((BLK,), lambda i: (i,)),
    out_shape=jax.ShapeDtypeStruct((N_TC,), jnp.float32),
)


def kernel(x, atom_type, shift, scale):
    xf = x.reshape(-1)
    t = atom_type.astype(jnp.int32)
    sc_out = _rescale_sc(xf, t, shift, scale)
    tc_out = _rescale_tc(xf, t, shift, scale)
    out = jnp.concatenate([tc_out, sc_out])
    return out.reshape(N_TOTAL, 1)


# speedup vs baseline: 1.3328x; 1.0080x over previous
import functools

import jax
import jax.numpy as jnp
from jax import lax
from jax.experimental import pallas as pl
from jax.experimental.pallas import tpu as pltpu
from jax.experimental.pallas import tpu_sc as plsc

N_TOTAL = 100000
L = 16
NS = 16

BLK = 60416
N_TC = 60416
N_SC = N_TOTAL - N_TC
CH = 2560

_mesh = plsc.VectorSubcoreMesh(core_axis_name="c", subcore_axis_name="s",
                               num_cores=1)


@functools.partial(
    pl.kernel,
    out_type=jax.ShapeDtypeStruct((N_SC,), jnp.float32),
    mesh=_mesh,
    scratch_types=[
        pltpu.VMEM((CH,), jnp.float32),
        pltpu.VMEM((CH,), jnp.int32),
        pltpu.VMEM((CH,), jnp.float32),
        pltpu.VMEM((L,), jnp.float32),
        pltpu.VMEM((L,), jnp.float32),
        pltpu.SemaphoreType.DMA,
        pltpu.SemaphoreType.DMA,
        pltpu.SemaphoreType.DMA,
    ],
)
def _rescale_sc(x_hbm, t_hbm, shift_hbm, scale_hbm, out_hbm,
                x_v, t_v, o_v, sc_v, sh_v, sem_tab, sem_x, sem_t):
    wid = lax.axis_index("s")
    base = jnp.minimum(wid * CH, N_SC - CH)

    c_sc = pltpu.async_copy(scale_hbm, sc_v, sem_tab)
    c_sh = pltpu.async_copy(shift_hbm, sh_v, sem_tab)
    c_x = pltpu.async_copy(x_hbm.at[pl.ds(N_TC + base, CH)], x_v, sem_x)
    c_t = pltpu.async_copy(t_hbm.at[pl.ds(N_TC + base, CH)], t_v, sem_t)
    c_sc.wait()
    c_sh.wait()
    scale_reg = sc_v[...]
    shift_reg = sh_v[...]
    c_x.wait()
    c_t.wait()

    @plsc.parallel_loop(0, CH, step=L, unroll=8)
    def _(off):
        t = t_v[pl.ds(off, L)]
        xv = x_v[pl.ds(off, L)]
        s = scale_reg.at[t].get(mode="promise_in_bounds")
        b = shift_reg.at[t].get(mode="promise_in_bounds")
        o_v[pl.ds(off, L)] = xv * s + b

    pltpu.sync_copy(o_v, out_hbm.at[pl.ds(base, CH)])


def _rescale_tc_body(x_ref, t_ref, shift_ref, scale_ref, o_ref):
    t = t_ref[...]
    xv = x_ref[...]
    s = jnp.full_like(xv, scale_ref[0])
    b = jnp.full_like(xv, shift_ref[0])
    for k in range(1, L):
        m = t == k
        s = jnp.where(m, scale_ref[k], s)
        b = jnp.where(m, shift_ref[k], b)
    o_ref[...] = xv * s + b


_rescale_tc = pl.pallas_call(
    _rescale_tc_body,
    grid=(N_TC // BLK,),
    in_specs=[
        pl.BlockSpec((BLK,), lambda i: (i,)),
        pl.BlockSpec((BLK,), lambda i: (i,)),
        pl.BlockSpec(memory_space=pltpu.SMEM),
        pl.BlockSpec(memory_space=pltpu.SMEM),
    ],
    out_specs=pl.BlockSpec((BLK,), lambda i: (i,)),
    out_shape=jax.ShapeDtypeStruct((N_TC,), jnp.float32),
)


def kernel(x, atom_type, shift, scale):
    xf = x.reshape(-1)
    t = atom_type.astype(jnp.int32)
    sc_out = _rescale_sc(xf, t, shift, scale)
    tc_out = _rescale_tc(xf, t, shift, scale)
    out = jnp.concatenate([tc_out, sc_out])
    return out.reshape(N_TOTAL, 1)
